# baseline (speedup 1.0000x reference)
import functools

import jax
import jax.numpy as jnp
from jax import lax
from jax.experimental import pallas as pl
from jax.experimental.pallas import tpu as pltpu
from jax.experimental.pallas import tpu_sc as plsc

_NUM_CODES = 8192
_DIM = 256
_BETA = 0.25
_ROWS = 16384

_R_TILE = 512
_C_TILE = 1024


def _norm_helper(x, axis=-1, eps=1e-12):
    n = jnp.sqrt(jnp.sum(x * x, axis=axis, keepdims=True))
    return x / jnp.maximum(n, eps)


def _rcp_body(x_ref, n_ref, o_ref):
    o_ref[...] = x_ref[...] * pl.reciprocal(n_ref[...], approx=True,
                                            full_range=False)


def _rcp_norm(x2d, n2d, blk):
    rows = x2d.shape[0]
    return pl.pallas_call(
        _rcp_body,
        grid=(rows // blk,),
        in_specs=[pl.BlockSpec((blk, x2d.shape[1]), lambda i: (i, 0)),
                  pl.BlockSpec((blk, 1), lambda i: (i, 0))],
        out_specs=pl.BlockSpec((blk, x2d.shape[1]), lambda i: (i, 0)),
        out_shape=jax.ShapeDtypeStruct(x2d.shape, jnp.float32),
    )(x2d, n2d)


def _argmin_body(z_ref, w_ref, zi2_ref, wn2_ref, idx_ref):
    zv = z_ref[...].astype(jnp.bfloat16)
    zi2 = zi2_ref[...]
    run_min = None
    run_idx = None
    for j in range(_NUM_CODES // _C_TILE):
        w_t = w_ref[pl.ds(j * _C_TILE, _C_TILE), :].astype(jnp.bfloat16)
        m = lax.dot_general(zv, w_t, (((1,), (1,)), ((), ())),
                            preferred_element_type=jnp.float32)
        d = (zi2 + wn2_ref[:, pl.ds(j * _C_TILE, _C_TILE)]) - 2.0 * m
        lm = jnp.min(d, axis=1, keepdims=True)
        la = (jnp.argmin(d, axis=1).astype(jnp.int32)
              .reshape(_R_TILE, 1) + j * _C_TILE)
        if j == 0:
            run_min, run_idx = lm, la
        else:
            upd = lm < run_min
            run_idx = jnp.where(upd, la, run_idx)
            run_min = jnp.where(upd, lm, run_min)
    idx_ref[...] = run_idx


def _argmin_call(z_flat, wn, zi2, wn2_row):
    grid = (_ROWS // _R_TILE,)
    return pl.pallas_call(
        _argmin_body,
        grid=grid,
        in_specs=[
            pl.BlockSpec((_R_TILE, _DIM), lambda i: (i, 0)),
            pl.BlockSpec((_NUM_CODES, _DIM), lambda i: (0, 0)),
            pl.BlockSpec((_R_TILE, 1), lambda i: (i, 0)),
            pl.BlockSpec((1, _NUM_CODES), lambda i: (0, 0)),
        ],
        out_specs=pl.BlockSpec((_R_TILE, 1), lambda i: (i, 0)),
        out_shape=jax.ShapeDtypeStruct((_ROWS, 1), jnp.int32),
        compiler_params=pltpu.CompilerParams(
            dimension_semantics=("arbitrary",)),
    )(z_flat, wn, zi2, wn2_row)


def _gather_call(table, idx):
    info = plsc.get_sparse_core_info()
    nc, ns = info.num_cores, info.num_subcores
    nw = nc * ns
    b_per_w = _ROWS // nw
    chunk = 128
    n_chunks = b_per_w // chunk
    mesh = plsc.VectorSubcoreMesh(core_axis_name="c", subcore_axis_name="s")

    @functools.partial(
        pl.kernel,
        mesh=mesh,
        out_type=jax.ShapeDtypeStruct((_ROWS, _DIM), jnp.float32),
        scratch_types=[
            pltpu.VMEM((b_per_w,), jnp.int32),
            pltpu.VMEM((chunk, _DIM), jnp.float32),
            pltpu.VMEM((chunk, _DIM), jnp.float32),
            pltpu.SemaphoreType.DMA,
            pltpu.SemaphoreType.DMA,
        ],
    )
    def _gather(table_hbm, idx_hbm, out_hbm, idx_v, buf0, buf1, sem0, sem1):
        wid = lax.axis_index("s") * nc + lax.axis_index("c")
        base = wid * b_per_w
        pltpu.sync_copy(idx_hbm.at[pl.ds(base, b_per_w)], idx_v)
        bufs = (buf0, buf1)
        sems = (sem0, sem1)
        handles = [None, None]
        handles[0] = pltpu.async_copy(
            table_hbm.at[idx_v.at[pl.ds(0, chunk)]], bufs[0], sems[0])
        for c in range(n_chunks):
            if c + 1 < n_chunks:
                handles[(c + 1) % 2] = pltpu.async_copy(
                    table_hbm.at[idx_v.at[pl.ds((c + 1) * chunk, chunk)]],
                    bufs[(c + 1) % 2], sems[(c + 1) % 2])
            handles[c % 2].wait()
            pltpu.sync_copy(bufs[c % 2],
                            out_hbm.at[pl.ds(base + c * chunk, chunk)])

    return _gather(table, idx)


_L_TILE = 1024


def _st_loss_body(zp_ref, zq_ref, out_ref, loss_ref):
    i = pl.program_id(0)
    zp = zp_ref[...]
    zq = zq_ref[...]
    diff = zq - zp
    out_ref[...] = zp + diff
    part = jnp.sum(diff * diff)
    loss_ref[0, 0] = jnp.where(i == 0, part, loss_ref[0, 0] + part)


def _st_loss_call(zp_flat, zq_flat):
    grid = (_ROWS // _L_TILE,)
    return pl.pallas_call(
        _st_loss_body,
        grid=grid,
        in_specs=[
            pl.BlockSpec((_L_TILE, _DIM), lambda i: (i, 0)),
            pl.BlockSpec((_L_TILE, _DIM), lambda i: (i, 0)),
        ],
        out_specs=[
            pl.BlockSpec((_L_TILE, _DIM), lambda i: (i, 0)),
            pl.BlockSpec(memory_space=pltpu.SMEM),
        ],
        out_shape=[
            jax.ShapeDtypeStruct((_ROWS, _DIM), jnp.float32),
            jax.ShapeDtypeStruct((1, 1), jnp.float32),
        ],
        compiler_params=pltpu.CompilerParams(
            dimension_semantics=("arbitrary",)),
    )(zp_flat, zq_flat)


def kernel(z, emb_weight):
    n = jnp.maximum(jnp.sqrt(jnp.sum(z * z, axis=-1, keepdims=True)), 1e-12)
    zn = _rcp_norm(z.reshape(-1, 32), n.reshape(-1, 1), 8192).reshape(z.shape)
    zp = jnp.transpose(zn, (0, 2, 3, 1))
    z_flat = zp.reshape(-1, _DIM)
    nw = jnp.maximum(jnp.sqrt(jnp.sum(emb_weight * emb_weight, axis=-1,
                                      keepdims=True)), 1e-12)
    wn = _rcp_norm(emb_weight, nw, 8192)
    zi2 = jnp.sum(z_flat ** 2, axis=1, keepdims=True)
    wn2 = jnp.sum(wn ** 2, axis=1)

    idx2d = _argmin_call(z_flat, wn, zi2, wn2.reshape(1, _NUM_CODES))
    idx = idx2d.reshape(_ROWS)

    zq_flat = _gather_call(emb_weight, idx)

    zq_st_flat, loss_sum = _st_loss_call(z_flat, zq_flat)

    m = loss_sum[0, 0] / jnp.float32(_ROWS * _DIM)
    loss = _BETA * m + m
    z_q = jnp.transpose(zq_st_flat.reshape(16, 32, 32, _DIM), (0, 3, 1, 2))
    return (z_q, idx, loss)
